# baseline (speedup 1.0000x reference)
import functools

import jax
import jax.numpy as jnp
from jax import lax
from jax.experimental import pallas as pl
from jax.experimental.pallas import tpu as pltpu
from jax.experimental.pallas import tpu_sc as plsc

_ROWS = 16384
_COLS = 128

_info = plsc.get_sparse_core_info()
_NC, _NS, _L = _info.num_cores, _info.num_subcores, _info.num_lanes
_NW = _NC * _NS
_RW = _ROWS // _NW
_WSZ = _RW * _COLS
_NGRP = _COLS // _L

_CH = 64
_CW = _CH * _COLS
_NCH = _RW // _CH

_mesh = plsc.VectorSubcoreMesh(core_axis_name="c", subcore_axis_name="s")


@functools.partial(
    pl.kernel,
    mesh=_mesh,
    out_type=jax.ShapeDtypeStruct((_ROWS * _COLS,), jnp.float32),
    scratch_types=[pltpu.VMEM((_CW,), jnp.float32)] * _NCH + [
        pltpu.VMEM((_COLS,), jnp.int32),
    ] + [pltpu.SemaphoreType.DMA] * (2 * _NCH + 1),
    compiler_params=pltpu.CompilerParams(
        needs_layout_passes=False,
        skip_device_barrier=True,
    ),
)
def _rearrange(x_hbm, order_hbm, out_hbm, *rest):
    inb = rest[:_NCH]
    idx_v = rest[_NCH]
    sems = rest[_NCH + 1:]
    wid = lax.axis_index("s") * _NC + lax.axis_index("c")
    base = wid * _WSZ
    isem, osem, qsem = sems[:_NCH], sems[_NCH:2 * _NCH], sems[2 * _NCH]

    def in_copy(c):
        return pltpu.make_async_copy(
            x_hbm.at[pl.ds(base + c * _CW, _CW)], inb[c], isem[c])

    def out_copy(c):
        return pltpu.make_async_copy(
            inb[c], out_hbm.at[pl.ds(base + c * _CW, _CW)], osem[c])

    for c in range(_NCH):
        in_copy(c).start()
    order_copy = pltpu.make_async_copy(order_hbm, idx_v, qsem)
    order_copy.start()
    order_copy.wait()
    gidx = [idx_v[pl.ds(_L * k, _L)] for k in range(_NGRP)]

    for c in range(_NCH):
        in_copy(c).wait()
        buf = inb[c]

        @plsc.parallel_loop(0, _CH, unroll=4)
        def _row(r):
            off = r * _COLS
            vals = [plsc.load_gather(buf, [off + gidx[k]])
                    for k in range(_NGRP)]
            for k in range(_NGRP):
                buf[pl.ds(off + _L * k, _L)] = vals[k]

        out_copy(c).start()

    for c in range(_NCH):
        out_copy(c).wait()


def kernel(x, order):
    out_flat = _rearrange(x.reshape(-1), order)
    return out_flat.reshape(_ROWS, _COLS)
